# baseline (speedup 1.0000x reference)
import jax
import jax.numpy as jnp
from jax.experimental import pallas as pl
from jax.experimental.pallas import tpu as pltpu


H0, W0 = 32, 50
WPAD = 56
HW = H0 * WPAD
ROW_PAD = 256
S = HW + ROW_PAD
CHUNK = 224


def _ceil_chunks(nrows):
    return -(-nrows // CHUNK) * CHUNK


_NEED_ROWS = dict(conv1=28, conv2=26, pool2=24, conv3=22,
                  pool3=20, conv4=16, pool4=12)
NROWS = {k: min(HW, _ceil_chunks((r + 1) * WPAD)) for k, r in _NEED_ROWS.items()}

NR_COMPACT = 13 * WPAD
NR_CONV4C = _ceil_chunks((8 + 1) * WPAD)
NR_POOL4C = _ceil_chunks((6 + 1) * WPAD)

NR_CONV1 = (_NEED_ROWS["conv1"] + 1) * WPAD


def _tap_groups(kh, kw, gsz):
    taps = list(range(kh * kw))
    return [taps[i:i + gsz] for i in range(0, len(taps), gsz)]


def _conv(src, dst, w_refs, b_ref, nrows, kin, nout, kh, kw, dhw, gsz, relu):
    groups = _tap_groups(kh, kw, gsz)
    w_mats = w_refs
    dh, dw = dhw
    bias = b_ref[...]
    for c0 in range(0, nrows, CHUNK):
        csz = min(CHUNK, nrows - c0)
        acc = None
        for g, wm in zip(groups, w_mats):
            parts = []
            for t in g:
                sh = dh * (t // kw) * WPAD + dw * (t % kw)
                parts.append(src[pl.ds(c0 + sh, csz), 0:kin])
            a = parts[0] if len(parts) == 1 else jnp.concatenate(parts, axis=1)
            p = jnp.dot(a, wm, preferred_element_type=jnp.float32)
            acc = p if acc is None else acc + p
        y = acc + bias
        if relu:
            y = jnp.maximum(y, 0.0)
        dst[pl.ds(c0, csz), 0:nout] = y.astype(jnp.bfloat16)


def _pool_relu(src, mid, dst, nrows, nl, kh, kw, dhw):
    dh, dw = dhw
    ext = _ceil_chunks(nrows + dh * (kh - 1) * WPAD)
    for c0 in range(0, ext, CHUNK):
        m = None
        for j in range(kw):
            v = src[pl.ds(c0 + dw * j, CHUNK), 0:nl]
            m = v if m is None else jnp.maximum(m, v)
        mid[pl.ds(c0, CHUNK), 0:nl] = m
    for c0 in range(0, nrows, CHUNK):
        m = None
        for i in range(kh):
            v = mid[pl.ds(c0 + dh * i * WPAD, CHUNK), 0:nl]
            m = v if m is None else jnp.maximum(m, v)
        dst[pl.ds(c0, CHUNK), 0:nl] = jnp.maximum(m, 0.0)


_GSZ = dict(conv1=4, conv2=2, conv3=1, conv4=1)


def _make_cnn_kernel(ns):
    n1 = len(_tap_groups(3, 3, _GSZ["conv1"]))
    n2 = len(_tap_groups(3, 3, _GSZ["conv2"]))

    def body(x_ref, *refs):
        w1 = [r[...] for r in refs[0:n1]]; b1 = refs[n1]
        w2 = [r[...] for r in refs[n1 + 1:n1 + 1 + n2]]; b2 = refs[n1 + 1 + n2]
        w3, b3, w4, b4 = refs[n1 + n2 + 2:n1 + n2 + 6]
        o_ref, buf_a, buf_b = refs[n1 + n2 + 6:]
        _conv(x_ref, buf_a, w1, b1, NR_CONV1, ns * 8, ns * 16,
              3, 3, (1, 1), _GSZ["conv1"], True)
        _conv(buf_a, buf_b, w2, b2, NROWS["conv2"], ns * 16, ns * 32,
              3, 3, (1, 1), _GSZ["conv2"], False)
        _pool_relu(buf_b, buf_a, buf_b, NROWS["pool2"], ns * 32, 3, 3, (1, 1))
        _conv(buf_b, buf_a, [w3[t] for t in range(15)], b3, NROWS["conv3"],
              ns * 32, ns * 32, 3, 5, (1, 1), _GSZ["conv3"], False)
        _pool_relu(buf_a, buf_b, buf_a, NROWS["pool3"], ns * 32, 3, 3, (1, 1))
        for hh in range(NR_COMPACT // WPAD):
            buf_b[pl.ds(hh * WPAD, WPAD), 0:ns * 64] = \
                buf_a[pl.ds(2 * hh * WPAD, WPAD), 0:ns * 64]
        _conv(buf_b, buf_a, [w4[t] for t in range(9)], b4, NR_CONV4C,
              ns * 32, ns * 64, 3, 3, (1, 2), _GSZ["conv4"], False)
        _pool_relu(buf_a, buf_b, buf_a, NR_POOL4C, ns * 64, 3, 3, (1, 2))
        for qh in range(4):
            for qw in range(8):
                p = 2 * qh * WPAD + 4 * qw
                q = qh * 8 + qw
                o_ref[q:q + 1, :] = buf_a[p:p + 1, 0:ns * 64]
    return body


def _mlp_kernel(l_ref, wl1_ref, bl1_ref, wl2_ref, bl2_ref, o_ref):
    h = jnp.dot(l_ref[...], wl1_ref[...],
                preferred_element_type=jnp.float32) + bl1_ref[...]
    h = jnp.maximum(h, 0.0).astype(jnp.bfloat16)
    o_ref[...] = jnp.dot(h, wl2_ref[...],
                         preferred_element_type=jnp.float32) + bl2_ref[...]


def _full_spec(arr):
    nd = arr.ndim
    return pl.BlockSpec(arr.shape, lambda g, _nd=nd: (0,) * _nd)


def _choose_ns(batch):
    for ns in (8, 4, 2, 1):
        if batch % ns == 0 and batch // ns >= 2:
            return ns
    return 1


@jax.jit
def _forward(w1, b1, w2, b2, w3, b3, w4, b4, wl1, bl1, wl2, bl2, x_nchw):
    B = x_nchw.shape[0]
    fw = wl2.shape[-1]
    ns = _choose_ns(B)
    nsteps = B // ns

    x = x_nchw.astype(jnp.bfloat16)
    x = jnp.pad(x, ((0, 0), (0, 2), (0, 0), (0, WPAD - W0)))
    x = x.reshape(nsteps, ns, 8, H0, WPAD).transpose(0, 3, 4, 1, 2)
    x_flat = x.reshape(nsteps * HW, ns * 8)

    eye = jnp.eye(ns, dtype=jnp.bfloat16)

    def bd(w):
        t, kin, kout = w.shape
        return jnp.einsum("ab,tkc->takbc", eye, w).reshape(t, ns * kin, ns * kout)

    def stacked(w, kh, kw, gsz):
        wbd = bd(w)
        return [jnp.concatenate([wbd[t] for t in g], axis=0)
                for g in _tap_groups(kh, kw, gsz)]

    w1g = stacked(w1, 3, 3, _GSZ["conv1"])
    w2g = stacked(w2, 3, 3, _GSZ["conv2"])
    wb3, wb4 = bd(w3), bd(w4)
    bb = [jnp.tile(b, (1, ns)) for b in (b1, b2, b3, b4)]

    operands = (x_flat, *w1g, bb[0], *w2g, bb[1], wb3, bb[2], wb4, bb[3])
    staged = pl.pallas_call(
        _make_cnn_kernel(ns),
        out_shape=jax.ShapeDtypeStruct((nsteps * 32, ns * 64), jnp.bfloat16),
        grid=(nsteps,),
        in_specs=[pl.BlockSpec((HW, ns * 8), lambda g: (g, 0))]
                 + [_full_spec(w) for w in operands[1:]],
        out_specs=pl.BlockSpec((32, ns * 64), lambda g: (g, 0)),
        scratch_shapes=[pltpu.VMEM((S, ns * 64), jnp.bfloat16),
                        pltpu.VMEM((S, ns * 64), jnp.bfloat16)],
        compiler_params=pltpu.CompilerParams(
            dimension_semantics=("parallel",)),
    )(*operands)

    lhs = staged.reshape(nsteps, 32, ns, 64).transpose(0, 2, 1, 3)
    lhs = lhs.reshape(B, 32 * 64)

    g2 = 4 if B % 4 == 0 else (2 if B % 2 == 0 else 1)
    out = pl.pallas_call(
        _mlp_kernel,
        out_shape=jax.ShapeDtypeStruct((B, fw), jnp.float32),
        grid=(g2,),
        in_specs=[pl.BlockSpec((B // g2, 32 * 64), lambda g: (g, 0)),
                  _full_spec(wl1), _full_spec(bl1),
                  _full_spec(wl2), _full_spec(bl2)],
        out_specs=pl.BlockSpec((B // g2, fw), lambda g: (g, 0)),
        compiler_params=pltpu.CompilerParams(
            dimension_semantics=("parallel",)),
    )(lhs, wl1, bl1, wl2, bl2)
    return out


def kernel(w1, b1, w2, b2, w3, b3, w4, b4, wl1, bl1, wl2, bl2, x_nchw):
    return _forward(w1, b1, w2, b2, w3, b3, w4, b4, wl1, bl1, wl2, bl2, x_nchw)


# speedup vs baseline: 1.0883x; 1.0883x over previous
import jax
import jax.numpy as jnp
from jax.experimental import pallas as pl
from jax.experimental.pallas import tpu as pltpu


H0, W0 = 32, 50
WPAD = 56
HW = H0 * WPAD
ROW_PAD = 256
S = HW + ROW_PAD
CHUNK = 224


def _ceil_chunks(nrows):
    return -(-nrows // CHUNK) * CHUNK


_NEED_ROWS = dict(conv1=28, conv2=26, pool2=24, conv3=22,
                  pool3=20, conv4=16, pool4=12)
NROWS = {k: min(HW, _ceil_chunks((r + 1) * WPAD)) for k, r in _NEED_ROWS.items()}

NR_COMPACT = 13 * WPAD
NR_CONV4C = _ceil_chunks((8 + 1) * WPAD)
NR_POOL4C = _ceil_chunks((6 + 1) * WPAD)

NR_CONV1 = (_NEED_ROWS["conv1"] + 1) * WPAD


def _tap_groups(kh, kw, gsz):
    taps = list(range(kh * kw))
    return [taps[i:i + gsz] for i in range(0, len(taps), gsz)]


def _conv(src, dst, w_refs, b_ref, nrows, kin, nout, kh, kw, dhw, gsz, relu):
    groups = _tap_groups(kh, kw, gsz)
    w_mats = w_refs
    dh, dw = dhw
    bias = b_ref[...]
    for c0 in range(0, nrows, CHUNK):
        csz = min(CHUNK, nrows - c0)
        acc = None
        for g, wm in zip(groups, w_mats):
            parts = []
            for t in g:
                sh = dh * (t // kw) * WPAD + dw * (t % kw)
                parts.append(src[pl.ds(c0 + sh, csz), 0:kin])
            a = parts[0] if len(parts) == 1 else jnp.concatenate(parts, axis=1)
            p = jnp.dot(a, wm, preferred_element_type=jnp.float32)
            acc = p if acc is None else acc + p
        y = acc + bias
        if relu:
            y = jnp.maximum(y, 0.0)
        dst[pl.ds(c0, csz), 0:nout] = y.astype(jnp.bfloat16)


def _pool_relu(src, mid, dst, nrows, nl, kh, kw, dhw):
    dh, dw = dhw
    ext = _ceil_chunks(nrows + dh * (kh - 1) * WPAD)
    for c0 in range(0, ext, CHUNK):
        m = None
        for j in range(kw):
            v = src[pl.ds(c0 + dw * j, CHUNK), 0:nl]
            m = v if m is None else jnp.maximum(m, v)
        mid[pl.ds(c0, CHUNK), 0:nl] = m
    for c0 in range(0, nrows, CHUNK):
        m = None
        for i in range(kh):
            v = mid[pl.ds(c0 + dh * i * WPAD, CHUNK), 0:nl]
            m = v if m is None else jnp.maximum(m, v)
        dst[pl.ds(c0, CHUNK), 0:nl] = jnp.maximum(m, 0.0)


_GSZ = dict(conv1=4, conv2=2, conv3=1, conv4=1)


def _make_cnn_kernel(ns):
    n1 = len(_tap_groups(3, 3, _GSZ["conv1"]))
    n2 = len(_tap_groups(3, 3, _GSZ["conv2"]))

    def body(x_ref, *refs):
        w1 = [r[...] for r in refs[0:n1]]; b1 = refs[n1]
        w2 = [r[...] for r in refs[n1 + 1:n1 + 1 + n2]]; b2 = refs[n1 + 1 + n2]
        w3, b3, w4, b4 = refs[n1 + n2 + 2:n1 + n2 + 6]
        o_ref, buf_a, buf_b = refs[n1 + n2 + 6:]
        _conv(x_ref, buf_a, w1, b1, NROWS["conv1"], ns * 8, ns * 16,
              3, 3, (1, 1), _GSZ["conv1"], True)
        _conv(buf_a, buf_b, w2, b2, NROWS["conv2"], ns * 16, ns * 32,
              3, 3, (1, 1), _GSZ["conv2"], False)
        _pool_relu(buf_b, buf_a, buf_b, NROWS["pool2"], ns * 32, 3, 3, (1, 1))
        _conv(buf_b, buf_a, [w3[t] for t in range(15)], b3, NROWS["conv3"],
              ns * 32, ns * 32, 3, 5, (1, 1), _GSZ["conv3"], False)
        _pool_relu(buf_a, buf_b, buf_a, NROWS["pool3"], ns * 32, 3, 3, (1, 1))
        for hh in range(NR_COMPACT // WPAD):
            buf_b[pl.ds(hh * WPAD, WPAD), 0:ns * 64] = \
                buf_a[pl.ds(2 * hh * WPAD, WPAD), 0:ns * 64]
        _conv(buf_b, buf_a, [w4[t] for t in range(9)], b4, NR_CONV4C,
              ns * 32, ns * 64, 3, 3, (1, 2), _GSZ["conv4"], False)
        _pool_relu(buf_a, buf_b, buf_a, NR_POOL4C, ns * 64, 3, 3, (1, 2))
        for qh in range(4):
            for qw in range(8):
                p = 2 * qh * WPAD + 4 * qw
                q = qh * 8 + qw
                o_ref[q:q + 1, :] = buf_a[p:p + 1, 0:ns * 64]
    return body


def _mlp_kernel(l_ref, wl1_ref, bl1_ref, wl2_ref, bl2_ref, o_ref):
    h = jnp.dot(l_ref[...], wl1_ref[...],
                preferred_element_type=jnp.float32) + bl1_ref[...]
    h = jnp.maximum(h, 0.0).astype(jnp.bfloat16)
    o_ref[...] = jnp.dot(h, wl2_ref[...],
                         preferred_element_type=jnp.float32) + bl2_ref[...]


def _full_spec(arr):
    nd = arr.ndim
    return pl.BlockSpec(arr.shape, lambda g, _nd=nd: (0,) * _nd)


def _choose_ns(batch):
    for ns in (8, 4, 2, 1):
        if batch % ns == 0 and batch // ns >= 2:
            return ns
    return 1


@jax.jit
def _forward(w1, b1, w2, b2, w3, b3, w4, b4, wl1, bl1, wl2, bl2, x_nchw):
    B = x_nchw.shape[0]
    fw = wl2.shape[-1]
    ns = _choose_ns(B)
    nsteps = B // ns

    x = x_nchw.astype(jnp.bfloat16)
    x = jnp.pad(x, ((0, 0), (0, 2), (0, 0), (0, WPAD - W0)))
    x = x.reshape(nsteps, ns, 8, H0, WPAD).transpose(0, 3, 4, 1, 2)
    x = x.reshape(nsteps, HW, ns * 8)
    x = jnp.pad(x, ((0, 0), (0, ROW_PAD), (0, 0)))
    x_flat = x.reshape(nsteps * S, ns * 8)

    eye = jnp.eye(ns, dtype=jnp.bfloat16)

    def bd(w):
        t, kin, kout = w.shape
        return jnp.einsum("ab,tkc->takbc", eye, w).reshape(t, ns * kin, ns * kout)

    def stacked(w, kh, kw, gsz):
        wbd = bd(w)
        return [jnp.concatenate([wbd[t] for t in g], axis=0)
                for g in _tap_groups(kh, kw, gsz)]

    w1g = stacked(w1, 3, 3, _GSZ["conv1"])
    w2g = stacked(w2, 3, 3, _GSZ["conv2"])
    wb3, wb4 = bd(w3), bd(w4)
    bb = [jnp.tile(b, (1, ns)) for b in (b1, b2, b3, b4)]

    operands = (x_flat, *w1g, bb[0], *w2g, bb[1], wb3, bb[2], wb4, bb[3])
    staged = pl.pallas_call(
        _make_cnn_kernel(ns),
        out_shape=jax.ShapeDtypeStruct((nsteps * 32, ns * 64), jnp.bfloat16),
        grid=(nsteps,),
        in_specs=[pl.BlockSpec((S, ns * 8), lambda g: (g, 0))]
                 + [_full_spec(w) for w in operands[1:]],
        out_specs=pl.BlockSpec((32, ns * 64), lambda g: (g, 0)),
        scratch_shapes=[pltpu.VMEM((S, ns * 64), jnp.bfloat16),
                        pltpu.VMEM((S, ns * 64), jnp.bfloat16)],
        compiler_params=pltpu.CompilerParams(
            dimension_semantics=("parallel",)),
    )(*operands)

    lhs = staged.reshape(nsteps, 32, ns, 64).transpose(0, 2, 1, 3)
    lhs = lhs.reshape(B, 32 * 64)

    g2 = 4 if B % 4 == 0 else (2 if B % 2 == 0 else 1)
    out = pl.pallas_call(
        _mlp_kernel,
        out_shape=jax.ShapeDtypeStruct((B, fw), jnp.float32),
        grid=(g2,),
        in_specs=[pl.BlockSpec((B // g2, 32 * 64), lambda g: (g, 0)),
                  _full_spec(wl1), _full_spec(bl1),
                  _full_spec(wl2), _full_spec(bl2)],
        out_specs=pl.BlockSpec((B // g2, fw), lambda g: (g, 0)),
        compiler_params=pltpu.CompilerParams(
            dimension_semantics=("parallel",)),
    )(lhs, wl1, bl1, wl2, bl2)
    return out


def kernel(w1, b1, w2, b2, w3, b3, w4, b4, wl1, bl1, wl2, bl2, x_nchw):
    return _forward(w1, b1, w2, b2, w3, b3, w4, b4, wl1, bl1, wl2, bl2, x_nchw)


# speedup vs baseline: 1.1766x; 1.0812x over previous
import jax
import jax.numpy as jnp
from jax.experimental import pallas as pl
from jax.experimental.pallas import tpu as pltpu


H0, W0 = 32, 50
WPAD = 56
HW = H0 * WPAD
ROW_PAD = 256
S = HW + ROW_PAD
CHUNK = 224


def _ceil_chunks(nrows):
    return -(-nrows // CHUNK) * CHUNK


_NEED_ROWS = dict(conv1=28, conv2=26, pool2=24, conv3=22,
                  pool3=20, conv4=16, pool4=12)
NROWS = {k: min(HW, _ceil_chunks((r + 1) * WPAD)) for k, r in _NEED_ROWS.items()}

WC = 32
NC_BANDS = 13
NC_BAND_W = 24
NR_CONV4C = (8 + 1) * WC
NR_POOL4C = (6 + 1) * WC

NR_CONV1 = (_NEED_ROWS["conv1"] + 1) * WPAD


def _tap_groups(kh, kw, gsz):
    taps = list(range(kh * kw))
    return [taps[i:i + gsz] for i in range(0, len(taps), gsz)]


def _conv(src, dst, w_refs, b_ref, nrows, kin, nout, kh, kw, dhw, gsz, relu,
          wpad=WPAD):
    groups = _tap_groups(kh, kw, gsz)
    w_mats = w_refs
    dh, dw = dhw
    bias = b_ref[...]
    for c0 in range(0, nrows, CHUNK):
        csz = min(CHUNK, nrows - c0)
        acc = None
        for g, wm in zip(groups, w_mats):
            parts = []
            for t in g:
                sh = dh * (t // kw) * wpad + dw * (t % kw)
                parts.append(src[pl.ds(c0 + sh, csz), 0:kin])
            a = parts[0] if len(parts) == 1 else jnp.concatenate(parts, axis=1)
            p = jnp.dot(a, wm, preferred_element_type=jnp.float32)
            acc = p if acc is None else acc + p
        y = acc + bias
        if relu:
            y = jnp.maximum(y, 0.0)
        dst[pl.ds(c0, csz), 0:nout] = y.astype(jnp.bfloat16)


def _pool_relu(src, mid, dst, nrows, nl, kh, kw, dhw, wpad=WPAD):
    dh, dw = dhw
    ext = _ceil_chunks(nrows + dh * (kh - 1) * wpad)
    for c0 in range(0, ext, CHUNK):
        m = None
        for j in range(kw):
            v = src[pl.ds(c0 + dw * j, CHUNK), 0:nl]
            m = v if m is None else jnp.maximum(m, v)
        mid[pl.ds(c0, CHUNK), 0:nl] = m
    for c0 in range(0, nrows, CHUNK):
        m = None
        for i in range(kh):
            v = mid[pl.ds(c0 + dh * i * wpad, CHUNK), 0:nl]
            m = v if m is None else jnp.maximum(m, v)
        dst[pl.ds(c0, CHUNK), 0:nl] = jnp.maximum(m, 0.0)


_GSZ = dict(conv1=4, conv2=2, conv3=1, conv4=1)


def _make_cnn_kernel(ns):
    n1 = len(_tap_groups(3, 3, _GSZ["conv1"]))
    n2 = len(_tap_groups(3, 3, _GSZ["conv2"]))

    def body(x_ref, *refs):
        w1 = [r[...] for r in refs[0:n1]]; b1 = refs[n1]
        w2 = [r[...] for r in refs[n1 + 1:n1 + 1 + n2]]; b2 = refs[n1 + 1 + n2]
        w3, b3, w4, b4 = refs[n1 + n2 + 2:n1 + n2 + 6]
        o_ref, buf_a, buf_b = refs[n1 + n2 + 6:]
        _conv(x_ref, buf_a, w1, b1, NROWS["conv1"], ns * 8, ns * 16,
              3, 3, (1, 1), _GSZ["conv1"], True)
        _conv(buf_a, buf_b, w2, b2, NROWS["conv2"], ns * 16, ns * 32,
              3, 3, (1, 1), _GSZ["conv2"], False)
        _pool_relu(buf_b, buf_a, buf_b, NROWS["pool2"], ns * 32, 3, 3, (1, 1))
        _conv(buf_b, buf_a, [w3[t] for t in range(15)], b3, NROWS["conv3"],
              ns * 32, ns * 32, 3, 5, (1, 1), _GSZ["conv3"], False)
        _pool_relu(buf_a, buf_b, buf_a, NROWS["pool3"], ns * 32, 3, 3, (1, 1))
        for hh in range(NC_BANDS):
            band = buf_a[pl.ds(2 * hh * WPAD, 2 * NC_BAND_W), 0:ns * 64]
            pairs = band.reshape(NC_BAND_W, 2 * ns * 64)
            buf_b[pl.ds(hh * WC, NC_BAND_W), 0:ns * 64] = pairs[:, 0:ns * 64]
        _conv(buf_b, buf_a, [w4[t] for t in range(9)], b4, NR_CONV4C,
              ns * 32, ns * 64, 3, 3, (1, 1), _GSZ["conv4"], False, wpad=WC)
        _pool_relu(buf_a, buf_b, buf_a, NR_POOL4C, ns * 64, 3, 3, (1, 1),
                   wpad=WC)
        for qh in range(4):
            for qw in range(8):
                p = 2 * qh * WC + 2 * qw
                q = qh * 8 + qw
                o_ref[q:q + 1, :] = buf_a[p:p + 1, 0:ns * 64]
    return body


def _mlp_kernel(l_ref, wl1_ref, bl1_ref, wl2_ref, bl2_ref, o_ref):
    h = jnp.dot(l_ref[...], wl1_ref[...],
                preferred_element_type=jnp.float32) + bl1_ref[...]
    h = jnp.maximum(h, 0.0).astype(jnp.bfloat16)
    o_ref[...] = jnp.dot(h, wl2_ref[...],
                         preferred_element_type=jnp.float32) + bl2_ref[...]


def _full_spec(arr):
    nd = arr.ndim
    return pl.BlockSpec(arr.shape, lambda g, _nd=nd: (0,) * _nd)


def _choose_ns(batch):
    for ns in (8, 4, 2, 1):
        if batch % ns == 0 and batch // ns >= 2:
            return ns
    return 1


@jax.jit
def _forward(w1, b1, w2, b2, w3, b3, w4, b4, wl1, bl1, wl2, bl2, x_nchw):
    B = x_nchw.shape[0]
    fw = wl2.shape[-1]
    ns = _choose_ns(B)
    nsteps = B // ns

    x = x_nchw.astype(jnp.bfloat16)
    x = jnp.pad(x, ((0, 0), (0, 2), (0, 0), (0, WPAD - W0)))
    x = x.reshape(nsteps, ns, 8, H0, WPAD).transpose(0, 3, 4, 1, 2)
    x = x.reshape(nsteps, HW, ns * 8)
    x = jnp.pad(x, ((0, 0), (0, ROW_PAD), (0, 0)))
    x_flat = x.reshape(nsteps * S, ns * 8)

    eye = jnp.eye(ns, dtype=jnp.bfloat16)

    def bd(w):
        t, kin, kout = w.shape
        return jnp.einsum("ab,tkc->takbc", eye, w).reshape(t, ns * kin, ns * kout)

    def stacked(w, kh, kw, gsz):
        wbd = bd(w)
        return [jnp.concatenate([wbd[t] for t in g], axis=0)
                for g in _tap_groups(kh, kw, gsz)]

    w1g = stacked(w1, 3, 3, _GSZ["conv1"])
    w2g = stacked(w2, 3, 3, _GSZ["conv2"])
    wb3, wb4 = bd(w3), bd(w4)
    bb = [jnp.tile(b, (1, ns)) for b in (b1, b2, b3, b4)]

    operands = (x_flat, *w1g, bb[0], *w2g, bb[1], wb3, bb[2], wb4, bb[3])
    staged = pl.pallas_call(
        _make_cnn_kernel(ns),
        out_shape=jax.ShapeDtypeStruct((nsteps * 32, ns * 64), jnp.bfloat16),
        grid=(nsteps,),
        in_specs=[pl.BlockSpec((S, ns * 8), lambda g: (g, 0))]
                 + [_full_spec(w) for w in operands[1:]],
        out_specs=pl.BlockSpec((32, ns * 64), lambda g: (g, 0)),
        scratch_shapes=[pltpu.VMEM((S, ns * 64), jnp.bfloat16),
                        pltpu.VMEM((S, ns * 64), jnp.bfloat16)],
        compiler_params=pltpu.CompilerParams(
            dimension_semantics=("parallel",)),
    )(*operands)

    lhs = staged.reshape(nsteps, 32, ns, 64).transpose(0, 2, 1, 3)
    lhs = lhs.reshape(B, 32 * 64)

    g2 = 4 if B % 4 == 0 else (2 if B % 2 == 0 else 1)
    out = pl.pallas_call(
        _mlp_kernel,
        out_shape=jax.ShapeDtypeStruct((B, fw), jnp.float32),
        grid=(g2,),
        in_specs=[pl.BlockSpec((B // g2, 32 * 64), lambda g: (g, 0)),
                  _full_spec(wl1), _full_spec(bl1),
                  _full_spec(wl2), _full_spec(bl2)],
        out_specs=pl.BlockSpec((B // g2, fw), lambda g: (g, 0)),
        compiler_params=pltpu.CompilerParams(
            dimension_semantics=("parallel",)),
    )(lhs, wl1, bl1, wl2, bl2)
    return out


def kernel(w1, b1, w2, b2, w3, b3, w4, b4, wl1, bl1, wl2, bl2, x_nchw):
    return _forward(w1, b1, w2, b2, w3, b3, w4, b4, wl1, bl1, wl2, bl2, x_nchw)
